# baseline (speedup 1.0000x reference)
import functools

import jax
import jax.numpy as jnp
from jax import lax
from jax.experimental import pallas as pl
from jax.experimental.pallas import tpu as pltpu
from jax.experimental.pallas import tpu_sc as plsc

_N = 64
_L = 16


@functools.cache
def _build_sigmoid_sc():
    mesh = plsc.VectorSubcoreMesh(
        core_axis_name="c", subcore_axis_name="s", num_cores=1, num_subcores=4
    )

    @functools.partial(
        pl.kernel,
        out_type=jax.ShapeDtypeStruct((_N,), jnp.float32),
        mesh=mesh,
        scratch_types=[pltpu.VMEM((_L,), jnp.float32)],
    )
    def _sigmoid_sc(prob_hbm, out_hbm, buf):
        sid = lax.axis_index("s")

        @pl.when(sid < _N // _L)
        def _():
            base = sid * _L
            pltpu.sync_copy(prob_hbm.at[pl.ds(base, _L)], buf)
            x = buf[...]
            buf[...] = 1.0 / (1.0 + jnp.exp(-x))
            pltpu.sync_copy(buf, out_hbm.at[pl.ds(base, _L)])

    return _sigmoid_sc


def kernel(prob):
    return _build_sigmoid_sc()(prob)
